# baseline (speedup 1.0000x reference)
import functools

import jax
import jax.numpy as jnp
from jax import lax
from jax.experimental import pallas as pl
from jax.experimental.pallas import tpu as pltpu
from jax.experimental.pallas import tpu_sc as plsc

EMB_D = 64
K = 1024
TPB = 1024
NB = 16
N_TOK = NB * TPB
COMMIT = 0.25


def _vq_tc_body(xc_ref, w_ref, wt_ref, xsq_ref, wsq_ref,
                idx_ref, q_ref, hist_ref, counts_ref, loss_ref, perp_ref,
                acc_counts, acc_loss):
    b = pl.program_id(0)
    xc = xc_ref[0]
    w = w_ref[...]
    wt = wt_ref[...]
    xsq = xsq_ref[0]
    wsq = wsq_ref[...]

    xw = jnp.dot(w, xc, preferred_element_type=jnp.float32)
    dist = (xsq + wsq) - 2.0 * xw

    minv = jnp.min(dist, axis=0, keepdims=True)
    sub = lax.broadcasted_iota(jnp.int32, (K, TPB), 0)
    idx = jnp.min(jnp.where(dist == minv, sub, K), axis=0, keepdims=True)
    idx_ref[0] = idx

    e = jnp.exp(minv - dist)
    s = jnp.sum(e, axis=0, keepdims=True)
    es = e * (1.0 / s)
    hist_ref[0] = jnp.sum(es, axis=1, keepdims=True)

    onehot = (sub == idx).astype(jnp.float32)
    counts_col = jnp.sum(onehot, axis=1, keepdims=True)
    counts_ref[0] = counts_col

    q_ref[0] = jnp.dot(wt, onehot, preferred_element_type=jnp.float32)

    lp = jnp.sum(minv, axis=1, keepdims=True)

    @pl.when(b == 0)
    def _():
        acc_counts[...] = counts_col
        acc_loss[...] = lp
        loss_ref[...] = jnp.zeros((1, 1), jnp.float32)
        perp_ref[...] = jnp.zeros((1, 1), jnp.float32)

    @pl.when(b > 0)
    def _():
        acc_counts[...] += counts_col
        acc_loss[...] += lp

    @pl.when(b == NB - 1)
    def _():
        avg = acc_counts[...] * (1.0 / N_TOK)
        ent = jnp.sum(avg * jnp.log(avg + 1e-10), axis=0, keepdims=True)
        perp_ref[...] = jnp.exp(-ent)
        loss_ref[...] = acc_loss[...] * ((1.0 + COMMIT) / (N_TOK * EMB_D))


def _build_tc(interpret=False):
    return pl.pallas_call(
        _vq_tc_body,
        grid=(NB,),
        in_specs=[
            pl.BlockSpec((1, EMB_D, TPB), lambda b: (b, 0, 0)),
            pl.BlockSpec((K, EMB_D), lambda b: (0, 0)),
            pl.BlockSpec((EMB_D, K), lambda b: (0, 0)),
            pl.BlockSpec((1, 1, TPB), lambda b: (b, 0, 0)),
            pl.BlockSpec((K, 1), lambda b: (0, 0)),
        ],
        out_specs=[
            pl.BlockSpec((1, 1, TPB), lambda b: (b, 0, 0)),
            pl.BlockSpec((1, EMB_D, TPB), lambda b: (b, 0, 0)),
            pl.BlockSpec((1, K, 1), lambda b: (b, 0, 0)),
            pl.BlockSpec((1, K, 1), lambda b: (b, 0, 0)),
            pl.BlockSpec((1, 1), lambda b: (0, 0)),
            pl.BlockSpec((1, 1), lambda b: (0, 0)),
        ],
        out_shape=[
            jax.ShapeDtypeStruct((NB, 1, TPB), jnp.int32),
            jax.ShapeDtypeStruct((NB, EMB_D, TPB), jnp.float32),
            jax.ShapeDtypeStruct((NB, K, 1), jnp.float32),
            jax.ShapeDtypeStruct((NB, K, 1), jnp.float32),
            jax.ShapeDtypeStruct((1, 1), jnp.float32),
            jax.ShapeDtypeStruct((1, 1), jnp.float32),
        ],
        scratch_shapes=[
            pltpu.VMEM((K, 1), jnp.float32),
            pltpu.VMEM((1, 1), jnp.float32),
        ],
        interpret=interpret,
    )


def _make_sc_quant():
    info = plsc.get_sparse_core_info()
    nc, ns = info.num_cores, info.num_subcores
    nw = nc * ns
    tok_per_w = N_TOK // nw
    groups = tok_per_w // 16
    mesh = plsc.VectorSubcoreMesh(core_axis_name="c", subcore_axis_name="s")

    @functools.partial(
        pl.kernel, mesh=mesh,
        out_type=jax.ShapeDtypeStruct((NB, EMB_D, TPB), jnp.float32),
        compiler_params=pltpu.CompilerParams(needs_layout_passes=False),
        scratch_types=[
            pltpu.VMEM((K * EMB_D,), jnp.float32),
            pltpu.VMEM((tok_per_w,), jnp.int32),
            pltpu.VMEM((EMB_D, tok_per_w), jnp.float32),
        ],
    )
    def quant_kernel(w_hbm, idx_hbm, out_hbm, w_v, idx_v, q_v):
        wid = lax.axis_index("c") * ns + lax.axis_index("s")
        base = wid * tok_per_w
        pltpu.sync_copy(w_hbm, w_v)
        pltpu.sync_copy(idx_hbm.at[pl.ds(base, tok_per_w)], idx_v)

        @plsc.parallel_loop(0, groups, 1)
        def body(g):
            start = pl.multiple_of(g * 16, 16)
            tix = idx_v[pl.ds(start, 16)] * EMB_D
            for d in range(EMB_D):
                q_v[d, pl.ds(start, 16)] = plsc.load_gather(w_v, [tix + d])
        b = base // TPB
        off = base % TPB
        pltpu.sync_copy(q_v, out_hbm.at[b, :, pl.ds(off, tok_per_w)])

    return quant_kernel


def kernel(input, W):
    xc = input.reshape(NB, EMB_D, TPB)
    x_fla = jnp.transpose(input, (0, 2, 3, 1)).reshape(N_TOK, EMB_D)
    xsq = jnp.sum(x_fla ** 2, axis=1, keepdims=True).reshape(NB, 1, TPB)
    wsq = jnp.sum(W ** 2, axis=1)[:, None]

    idx, q, hist_t, counts_t, loss, perp = _build_tc()(xc, W, W.T, xsq, wsq)
    q_sc = _make_sc_quant()(W.reshape(K * EMB_D), idx.reshape(N_TOK))
    quantized_out = q_sc.reshape(NB, EMB_D, 32, 32)

    return (quantized_out, loss[0, 0], perp[0, 0], idx.reshape(NB, TPB),
            counts_t.reshape(NB, K), hist_t.reshape(NB, K))


# speedup vs baseline: 1.4886x; 1.4886x over previous
import functools

import jax
import jax.numpy as jnp
from jax import lax
from jax.experimental import pallas as pl
from jax.experimental.pallas import tpu as pltpu
from jax.experimental.pallas import tpu_sc as plsc

EMB_D = 64
K = 1024
TPB = 1024
NB = 16
N_TOK = NB * TPB
COMMIT = 0.25


def _vq_tc_body(xc_ref, w_ref,
                idx_ref, q_ref, hist_ref, counts_ref, loss_ref, perp_ref,
                acc_counts, acc_loss):
    b = pl.program_id(0)
    xc = xc_ref[0]
    w = w_ref[...]
    xsq = jnp.sum(xc * xc, axis=0, keepdims=True)
    wsq = jnp.sum(w * w, axis=1, keepdims=True)

    xw = jnp.dot(w, xc, preferred_element_type=jnp.float32)
    dist = (xsq + wsq) - 2.0 * xw

    minv = jnp.min(dist, axis=0, keepdims=True)
    sub = lax.broadcasted_iota(jnp.int32, (K, TPB), 0)
    idx = jnp.min(jnp.where(dist == minv, sub, K), axis=0, keepdims=True)
    idx_ref[0] = idx

    e = jnp.exp(minv - dist)
    s = jnp.sum(e, axis=0, keepdims=True)
    es = e * (1.0 / s)
    hist_ref[0] = jnp.sum(es, axis=1, keepdims=True)

    onehot = (sub == idx).astype(jnp.float32)
    counts_col = jnp.sum(onehot, axis=1, keepdims=True)
    counts_ref[0] = counts_col

    q_ref[0] = lax.dot_general(w, onehot, (((0,), (0,)), ((), ())),
                               preferred_element_type=jnp.float32)

    lp = jnp.sum(minv, axis=1, keepdims=True)

    @pl.when(b == 0)
    def _():
        acc_counts[...] = counts_col
        acc_loss[...] = lp
        loss_ref[...] = jnp.zeros((1, 1), jnp.float32)
        perp_ref[...] = jnp.zeros((1, 1), jnp.float32)

    @pl.when(b > 0)
    def _():
        acc_counts[...] += counts_col
        acc_loss[...] += lp

    @pl.when(b == NB - 1)
    def _():
        avg = acc_counts[...] * (1.0 / N_TOK)
        ent = jnp.sum(avg * jnp.log(avg + 1e-10), axis=0, keepdims=True)
        perp_ref[...] = jnp.exp(-ent)
        loss_ref[...] = acc_loss[...] * ((1.0 + COMMIT) / (N_TOK * EMB_D))


def _build_tc(interpret=False):
    return pl.pallas_call(
        _vq_tc_body,
        grid=(NB,),
        in_specs=[
            pl.BlockSpec((1, EMB_D, TPB), lambda b: (b, 0, 0)),
            pl.BlockSpec((K, EMB_D), lambda b: (0, 0)),
        ],
        out_specs=[
            pl.BlockSpec((1, 1, TPB), lambda b: (b, 0, 0)),
            pl.BlockSpec((1, EMB_D, TPB), lambda b: (b, 0, 0)),
            pl.BlockSpec((1, K, 1), lambda b: (b, 0, 0)),
            pl.BlockSpec((1, K, 1), lambda b: (b, 0, 0)),
            pl.BlockSpec((1, 1), lambda b: (0, 0)),
            pl.BlockSpec((1, 1), lambda b: (0, 0)),
        ],
        out_shape=[
            jax.ShapeDtypeStruct((NB, 1, TPB), jnp.int32),
            jax.ShapeDtypeStruct((NB, EMB_D, TPB), jnp.float32),
            jax.ShapeDtypeStruct((NB, K, 1), jnp.float32),
            jax.ShapeDtypeStruct((NB, K, 1), jnp.float32),
            jax.ShapeDtypeStruct((1, 1), jnp.float32),
            jax.ShapeDtypeStruct((1, 1), jnp.float32),
        ],
        scratch_shapes=[
            pltpu.VMEM((K, 1), jnp.float32),
            pltpu.VMEM((1, 1), jnp.float32),
        ],
        interpret=interpret,
    )


def kernel(input, W):
    xc = input.reshape(NB, EMB_D, TPB)
    idx, q, hist_t, counts_t, loss, perp = _build_tc()(xc, W)
    quantized_out = q.reshape(NB, EMB_D, 32, 32)

    return (quantized_out, loss[0, 0], perp[0, 0], idx.reshape(NB, TPB),
            counts_t.reshape(NB, K), hist_t.reshape(NB, K))


# speedup vs baseline: 1.5164x; 1.0187x over previous
import functools

import jax
import jax.numpy as jnp
from jax import lax
from jax.experimental import pallas as pl
from jax.experimental.pallas import tpu as pltpu
from jax.experimental.pallas import tpu_sc as plsc

EMB_D = 64
K = 1024
TPB = 1024
NB = 16
N_TOK = NB * TPB
COMMIT = 0.25


def _vq_tc_body(xc_ref, w_ref,
                idx_ref, q_ref, hist_ref, counts_ref, loss_ref, perp_ref,
                acc_counts, acc_loss):
    b = pl.program_id(0)
    xc = xc_ref[0]
    w = w_ref[...]
    xsq = jnp.sum(xc * xc, axis=0, keepdims=True)
    wsq = jnp.sum(w * w, axis=1, keepdims=True)

    xw = jnp.dot(w, xc, preferred_element_type=jnp.float32)
    dist = (xsq + wsq) - 2.0 * xw

    minv = jnp.min(dist, axis=0, keepdims=True)
    sub = lax.broadcasted_iota(jnp.int32, (K, TPB), 0)
    idx = jnp.min(jnp.where(dist == minv, sub, K), axis=0, keepdims=True)
    idx_ref[pl.ds(b, 1), :] = idx

    e = jnp.exp(minv - dist)
    s = jnp.sum(e, axis=0, keepdims=True)
    es = e * (1.0 / s)
    hist_ref[0] = jnp.sum(es, axis=1, keepdims=True)

    onehot = (sub == idx).astype(jnp.float32)
    counts_col = jnp.sum(onehot, axis=1, keepdims=True)
    counts_ref[0] = counts_col

    q_ref[0] = lax.dot_general(w, onehot, (((0,), (0,)), ((), ())),
                               preferred_element_type=jnp.float32)

    lp = jnp.sum(minv, axis=1, keepdims=True)

    @pl.when(b == 0)
    def _():
        acc_counts[...] = counts_col
        acc_loss[...] = lp

    @pl.when(b > 0)
    def _():
        acc_counts[...] += counts_col
        acc_loss[...] += lp

    @pl.when(b == NB - 1)
    def _():
        avg = acc_counts[...] * (1.0 / N_TOK)
        ent = jnp.sum(avg * jnp.log(avg + 1e-10), axis=0, keepdims=True)
        perp_ref[0] = jnp.exp(-ent)[0, 0]
        loss_ref[0] = acc_loss[0, 0] * ((1.0 + COMMIT) / (N_TOK * EMB_D))


def _build_tc(interpret=False):
    return pl.pallas_call(
        _vq_tc_body,
        grid=(NB,),
        in_specs=[
            pl.BlockSpec((1, EMB_D, TPB), lambda b: (b, 0, 0)),
            pl.BlockSpec((K, EMB_D), lambda b: (0, 0)),
        ],
        out_specs=[
            pl.BlockSpec((NB, TPB), lambda b: (0, 0)),
            pl.BlockSpec((1, EMB_D, TPB), lambda b: (b, 0, 0)),
            pl.BlockSpec((1, K, 1), lambda b: (b, 0, 0)),
            pl.BlockSpec((1, K, 1), lambda b: (b, 0, 0)),
            pl.BlockSpec(memory_space=pltpu.SMEM),
            pl.BlockSpec(memory_space=pltpu.SMEM),
        ],
        out_shape=[
            jax.ShapeDtypeStruct((NB, TPB), jnp.int32),
            jax.ShapeDtypeStruct((NB, EMB_D, TPB), jnp.float32),
            jax.ShapeDtypeStruct((NB, K, 1), jnp.float32),
            jax.ShapeDtypeStruct((NB, K, 1), jnp.float32),
            jax.ShapeDtypeStruct((1,), jnp.float32),
            jax.ShapeDtypeStruct((1,), jnp.float32),
        ],
        scratch_shapes=[
            pltpu.VMEM((K, 1), jnp.float32),
            pltpu.VMEM((1, 1), jnp.float32),
        ],
        interpret=interpret,
    )


def kernel(input, W):
    xc = input.reshape(NB, EMB_D, TPB)
    idx, q, hist_t, counts_t, loss, perp = _build_tc()(xc, W)
    quantized_out = q.reshape(NB, EMB_D, 32, 32)

    return (quantized_out, loss[0], perp[0], idx,
            counts_t.reshape(NB, K), hist_t.reshape(NB, K))
